# baseline (speedup 1.0000x reference)
import jax
import jax.numpy as jnp
from jax.experimental import pallas as pl
from jax.experimental.pallas import tpu as pltpu
from jax.experimental.pallas import tpu_sc as plsc

_CHUNK = 512


def kernel(tokens, table):
    b, s = tokens.shape
    n = b * s
    emb = table.shape[1]
    idx = tokens.reshape(1, n).astype(jnp.int32)

    info = plsc.get_sparse_core_info()
    nw = info.num_cores * info.num_subcores
    per_w = n // nw
    n_chunks = per_w // _CHUNK
    assert per_w % _CHUNK == 0

    mesh = plsc.VectorSubcoreMesh(
        core_axis_name="core", subcore_axis_name="subcore"
    )

    @pl.kernel(
        out_type=jax.ShapeDtypeStruct((n, emb), table.dtype),
        mesh=mesh,
        compiler_params=pltpu.CompilerParams(use_tc_tiling_on_sc=False),
        scratch_types=[
            pltpu.VMEM((per_w,), jnp.int32),
            pltpu.VMEM((2, _CHUNK, 64), jnp.float32),
            pltpu.SemaphoreType.DMA((2,)),
            pltpu.SemaphoreType.DMA((2,)),
        ],
    )
    def gather_kernel(table_hbm, idx_hbm, out_hbm, idx_v, rows_v, gsem, ssem):
        wid = jax.lax.axis_index("subcore") * info.num_cores + jax.lax.axis_index(
            "core"
        )
        base = wid * per_w
        pltpu.sync_copy(idx_hbm.at[0, pl.ds(base, per_w)], idx_v)

        def gather_start(g, buf):
            pltpu.async_copy(
                table_hbm.at[idx_v.at[pl.ds(g * _CHUNK, _CHUNK)]],
                rows_v.at[buf],
                gsem.at[buf],
            )

        def gather_wait(buf):
            pltpu.make_async_copy(
                table_hbm.at[idx_v.at[pl.ds(0, _CHUNK)]],
                rows_v.at[buf],
                gsem.at[buf],
            ).wait()

        def store_start(g, buf):
            pltpu.async_copy(
                rows_v.at[buf],
                out_hbm.at[pl.ds(base + g * _CHUNK, _CHUNK)],
                ssem.at[buf],
            )

        def store_wait(buf):
            pltpu.make_async_copy(
                rows_v.at[buf],
                out_hbm.at[pl.ds(base, _CHUNK)],
                ssem.at[buf],
            ).wait()

        gather_start(0, 0)

        @pl.loop(0, n_chunks // 2)
        def _(h):
            for p in range(2):
                g = h * 2 + p
                buf = p
                gather_wait(buf)
                @pl.when(g + 1 < n_chunks)
                def _():
                    @pl.when(g >= 1)
                    def _():
                        store_wait(1 - buf)

                    gather_start(g + 1, 1 - buf)

                store_start(g, buf)

        for buf in range(2):
            store_wait(buf)

    out = gather_kernel(table, idx)
    return out.reshape(b, s, emb)
